# baseline (speedup 1.0000x reference)
import functools
import math

import jax
import jax.numpy as jnp
import numpy as np
from jax import lax
from jax.experimental import pallas as pl
from jax.experimental.pallas import tpu as pltpu
from jax.experimental.pallas import tpu_sc as plsc

_D_MODEL = 1024
_SEQ_LEN = 2048
_BATCH = 4
_MAX_LEN = 5000

_NC = 2
_NS = 16
_NW = _NC * _NS

_POS_PER_W = _SEQ_LEN // _NW
_CHUNK = 16
_NCHUNK = _POS_PER_W // _CHUNK
_NSTEP = _NCHUNK * _BATCH
_VECS_PER_ROW = _D_MODEL // 16


def _make_pe(d_model, seq_len):
    position = np.arange(_MAX_LEN)[:, np.newaxis]
    div_term = np.exp(np.arange(0, d_model, 2) * (-math.log(10000.0) / d_model))
    pe = np.zeros((_MAX_LEN, d_model))
    pe[:, 0::2] = np.sin(position * div_term)
    pe[:, 1::2] = np.cos(position * div_term)
    return pe[:seq_len].astype(np.float32)


def _pack_pe_words(pe):
    import ml_dtypes
    seq_len, d_model = pe.shape
    bits = pe.astype(ml_dtypes.bfloat16).view(np.uint16).astype(np.uint32)
    g = bits.reshape(seq_len, d_model // 32, 2, 16)
    words = g[:, :, 0, :] | (g[:, :, 1, :] << 16)
    return np.ascontiguousarray(
        words.reshape(seq_len, d_model // 2)).view(np.int32)


_PE_W = _pack_pe_words(_make_pe(_D_MODEL, _SEQ_LEN))


@functools.partial(
    pl.kernel,
    mesh=plsc.VectorSubcoreMesh(core_axis_name="c", subcore_axis_name="s"),
    out_type=jax.ShapeDtypeStruct((_BATCH * _SEQ_LEN, _D_MODEL), jnp.float32),
    scratch_types=[
        pltpu.VMEM((_NSTEP, _CHUNK), jnp.int32),
        pltpu.VMEM((_CHUNK, _D_MODEL), jnp.float32),
        pltpu.VMEM((_CHUNK, _D_MODEL), jnp.float32),
        pltpu.VMEM((_CHUNK, _D_MODEL), jnp.float32),
        pltpu.VMEM((_CHUNK, _D_MODEL), jnp.float32),
        pltpu.VMEM((_CHUNK, _D_MODEL), jnp.float32),
        pltpu.VMEM((_CHUNK, _D_MODEL), jnp.float32),
        pltpu.VMEM((_CHUNK, _D_MODEL // 2), jnp.int32),
        pltpu.VMEM((_CHUNK, _D_MODEL // 2), jnp.int32),
        pltpu.SemaphoreType.DMA,
        pltpu.SemaphoreType.DMA,
        pltpu.SemaphoreType.DMA,
        pltpu.SemaphoreType.DMA,
        pltpu.SemaphoreType.DMA,
        pltpu.SemaphoreType.DMA,
        pltpu.SemaphoreType.DMA,
        pltpu.SemaphoreType.DMA,
        pltpu.SemaphoreType.DMA,
        pltpu.SemaphoreType.DMA,
    ],
)
def _sc_embed(idx_hbm, table_hbm, pe_hbm, out_hbm,
              idx_v, r0, r1, r2, r3, p0, p1, pk0, pk1,
              gs0, gs1, gs2, gs3, os0, os1, os2, os3, ps0, ps1):
    wid = lax.axis_index("s") * _NC + lax.axis_index("c")
    pos0 = wid * _POS_PER_W

    rbuf = (r0, r1, r2, r3)
    pbuf = (p0, p1)
    pkbuf = (pk0, pk1)
    gsem = (gs0, gs1, gs2, gs3)
    osem = (os0, os1, os2, os3)
    psem = (ps0, ps1)
    nbuf = 4
    lead = 3

    pltpu.sync_copy(idx_hbm.at[wid], idx_v)

    pe_cp = [None, None]
    pe_cp[0] = pltpu.async_copy(pe_hbm.at[pl.ds(pos0, _CHUNK)], pk0, ps0)
    g_cp = [None] * _NSTEP
    o_cp = [None] * _NSTEP
    for t in range(lead):
        g_cp[t] = pltpu.async_copy(
            table_hbm.at[idx_v.at[t]], rbuf[t], gsem[t])

    def expand_pe(ci):
        pk = pkbuf[ci % 2]
        pe = pbuf[ci % 2]
        pe_cp[ci % 2].wait()

        def _exp_row(r, _):
            for j in range(_D_MODEL // 32):
                w = pk[r, pl.ds(j * 16, 16)]
                lo = lax.bitcast_convert_type(w << 16, jnp.float32)
                hi = lax.bitcast_convert_type(w & jnp.int32(-65536),
                                              jnp.float32)
                pe[r, pl.ds(j * 32, 16)] = lo
                pe[r, pl.ds(j * 32 + 16, 16)] = hi
            return 0

        lax.fori_loop(0, _CHUNK, _exp_row, 0)

    expand_pe(0)

    for s in range(_NSTEP):
        c, b = divmod(s, _BATCH)
        g_cp[s].wait()

        rb = rbuf[s % nbuf]
        pb = pbuf[c % 2]

        def _add_row(r, _):
            for k in range(_VECS_PER_ROW):
                sl = pl.ds(k * 16, 16)
                rb[r, sl] = rb[r, sl] + pb[r, sl]
            return 0

        lax.fori_loop(0, _CHUNK, _add_row, 0)
        o_cp[s] = pltpu.async_copy(
            rb, out_hbm.at[pl.ds(b * _SEQ_LEN + pos0 + c * _CHUNK, _CHUNK)],
            osem[s % nbuf])

        t = s + lead
        if t < _NSTEP:
            if t - nbuf >= 0:
                o_cp[t - nbuf].wait()
            c1, b1 = divmod(t, _BATCH)
            if b1 == 0:
                pe_cp[c1 % 2] = pltpu.async_copy(
                    pe_hbm.at[pl.ds(pos0 + c1 * _CHUNK, _CHUNK)],
                    pkbuf[c1 % 2], psem[c1 % 2])
            g_cp[t] = pltpu.async_copy(
                table_hbm.at[idx_v.at[t]], rbuf[t % nbuf], gsem[t % nbuf])

        if b == _BATCH - 1 and c + 1 < _NCHUNK:
            expand_pe(c + 1)

    for s in range(_NSTEP - nbuf, _NSTEP):
        o_cp[s].wait()


def kernel(x, emb_table):
    batch, seq_len = x.shape
    d_model = emb_table.shape[1]
    idx = (x.astype(jnp.int32)
           .reshape(batch, _NW, _NCHUNK, _CHUNK)
           .transpose(1, 2, 0, 3)
           .reshape(_NW, _NSTEP, _CHUNK))
    out = _sc_embed(idx, emb_table, jnp.asarray(_PE_W))
    return out.reshape(batch, seq_len, d_model)


# speedup vs baseline: 1.1004x; 1.1004x over previous
import functools
import math

import jax
import jax.numpy as jnp
import numpy as np
from jax import lax
from jax.experimental import pallas as pl
from jax.experimental.pallas import tpu as pltpu
from jax.experimental.pallas import tpu_sc as plsc

_D_MODEL = 1024
_SEQ_LEN = 2048
_BATCH = 4
_MAX_LEN = 5000

_NC = 2
_NS = 16
_NW = _NC * _NS

_POS_PER_W = _SEQ_LEN // _NW
_CHUNK = 16
_NCHUNK = _POS_PER_W // _CHUNK
_NSTEP = _NCHUNK * _BATCH
_VECS_PER_ROW = _D_MODEL // 16


def _make_pe(d_model, seq_len):
    position = np.arange(_MAX_LEN)[:, np.newaxis]
    div_term = np.exp(np.arange(0, d_model, 2) * (-math.log(10000.0) / d_model))
    pe = np.zeros((_MAX_LEN, d_model))
    pe[:, 0::2] = np.sin(position * div_term)
    pe[:, 1::2] = np.cos(position * div_term)
    return pe[:seq_len].astype(np.float32)


_PE = _make_pe(_D_MODEL, _SEQ_LEN)


@functools.partial(
    pl.kernel,
    mesh=plsc.VectorSubcoreMesh(core_axis_name="c", subcore_axis_name="s"),
    out_type=jax.ShapeDtypeStruct((_BATCH * _SEQ_LEN, _D_MODEL), jnp.float32),
    scratch_types=[
        pltpu.VMEM((_NSTEP, _CHUNK), jnp.int32),
        pltpu.VMEM((_CHUNK, _D_MODEL), jnp.float32),
        pltpu.VMEM((_CHUNK, _D_MODEL), jnp.float32),
        pltpu.VMEM((_CHUNK, _D_MODEL), jnp.float32),
        pltpu.VMEM((_CHUNK, _D_MODEL), jnp.float32),
        pltpu.VMEM((_CHUNK, _D_MODEL), jnp.float32),
        pltpu.VMEM((_CHUNK, _D_MODEL), jnp.float32),
        pltpu.SemaphoreType.DMA,
        pltpu.SemaphoreType.DMA,
        pltpu.SemaphoreType.DMA,
        pltpu.SemaphoreType.DMA,
        pltpu.SemaphoreType.DMA,
        pltpu.SemaphoreType.DMA,
        pltpu.SemaphoreType.DMA,
        pltpu.SemaphoreType.DMA,
        pltpu.SemaphoreType.DMA,
        pltpu.SemaphoreType.DMA,
    ],
)
def _sc_embed(idx_hbm, table_hbm, pe_hbm, out_hbm,
              idx_v, r0, r1, r2, r3, p0, p1,
              gs0, gs1, gs2, gs3, os0, os1, os2, os3, ps0, ps1):
    wid = lax.axis_index("s") * _NC + lax.axis_index("c")
    pos0 = wid * _POS_PER_W

    rbuf = (r0, r1, r2, r3)
    pbuf = (p0, p1)
    gsem = (gs0, gs1, gs2, gs3)
    osem = (os0, os1, os2, os3)
    psem = (ps0, ps1)
    nbuf = 4

    pltpu.sync_copy(idx_hbm.at[wid], idx_v)

    pe_cp = [None, None]
    pe_cp[0] = pltpu.async_copy(pe_hbm.at[pl.ds(pos0, _CHUNK)], p0, ps0)
    g_cp = [None] * _NSTEP
    o_cp = [None] * _NSTEP
    for t in range(nbuf - 1):
        g_cp[t] = pltpu.async_copy(
            table_hbm.at[idx_v.at[t]], rbuf[t], gsem[t])

    for s in range(_NSTEP):
        c, b = divmod(s, _BATCH)
        g_cp[s].wait()
        if b == 0:
            pe_cp[c % 2].wait()

        rb = rbuf[s % nbuf]
        pb = pbuf[c % 2]

        def _add_row(r, _):
            for k in range(_VECS_PER_ROW):
                sl = pl.ds(k * 16, 16)
                rb[r, sl] = rb[r, sl] + pb[r, sl]
            return 0

        lax.fori_loop(0, _CHUNK, _add_row, 0)
        o_cp[s] = pltpu.async_copy(
            rb, out_hbm.at[pl.ds(b * _SEQ_LEN + pos0 + c * _CHUNK, _CHUNK)],
            osem[s % nbuf])

        t = s + nbuf - 1
        if t < _NSTEP:
            if s >= 1:
                o_cp[s - 1].wait()
            c1, b1 = divmod(t, _BATCH)
            if b1 == 0:
                pe_cp[c1 % 2] = pltpu.async_copy(
                    pe_hbm.at[pl.ds(pos0 + c1 * _CHUNK, _CHUNK)],
                    pbuf[c1 % 2], psem[c1 % 2])
            g_cp[t] = pltpu.async_copy(
                table_hbm.at[idx_v.at[t]], rbuf[t % nbuf], gsem[t % nbuf])

    for s in range(_NSTEP - nbuf, _NSTEP):
        o_cp[s].wait()


def kernel(x, emb_table):
    batch, seq_len = x.shape
    d_model = emb_table.shape[1]
    idx = (x.astype(jnp.int32)
           .reshape(batch, _NW, _NCHUNK, _CHUNK)
           .transpose(1, 2, 0, 3)
           .reshape(_NW, _NSTEP, _CHUNK))
    out = _sc_embed(idx, emb_table, jnp.asarray(_PE))
    return out.reshape(batch, seq_len, d_model)


# speedup vs baseline: 1.1319x; 1.0287x over previous
import functools
import math

import jax
import jax.numpy as jnp
import numpy as np
from jax import lax
from jax.experimental import pallas as pl
from jax.experimental.pallas import tpu as pltpu
from jax.experimental.pallas import tpu_sc as plsc

_D_MODEL = 1024
_SEQ_LEN = 2048
_BATCH = 4
_MAX_LEN = 5000

_NC = 2
_NS = 16
_NW = _NC * _NS

_POS_PER_W = _SEQ_LEN // _NW
_CHUNK = 16
_NCHUNK = _POS_PER_W // _CHUNK
_NSTEP = _NCHUNK * _BATCH
_VECS_PER_ROW = _D_MODEL // 16


def _make_pe(d_model, seq_len):
    position = np.arange(_MAX_LEN)[:, np.newaxis]
    div_term = np.exp(np.arange(0, d_model, 2) * (-math.log(10000.0) / d_model))
    pe = np.zeros((_MAX_LEN, d_model))
    pe[:, 0::2] = np.sin(position * div_term)
    pe[:, 1::2] = np.cos(position * div_term)
    return pe[:seq_len].astype(np.float32)


_PE = _make_pe(_D_MODEL, _SEQ_LEN)


@functools.partial(
    pl.kernel,
    mesh=plsc.VectorSubcoreMesh(core_axis_name="c", subcore_axis_name="s"),
    out_type=jax.ShapeDtypeStruct((_BATCH * _SEQ_LEN, _D_MODEL), jnp.float32),
    scratch_types=[
        pltpu.VMEM((_NSTEP, _CHUNK), jnp.int32),
        pltpu.VMEM((_CHUNK, _D_MODEL), jnp.float32),
        pltpu.VMEM((_CHUNK, _D_MODEL), jnp.float32),
        pltpu.VMEM((_CHUNK, _D_MODEL), jnp.float32),
        pltpu.VMEM((_CHUNK, _D_MODEL), jnp.float32),
        pltpu.VMEM((_CHUNK, _D_MODEL), jnp.float32),
        pltpu.VMEM((_CHUNK, _D_MODEL), jnp.float32),
        pltpu.SemaphoreType.DMA,
        pltpu.SemaphoreType.DMA,
        pltpu.SemaphoreType.DMA,
        pltpu.SemaphoreType.DMA,
        pltpu.SemaphoreType.DMA,
        pltpu.SemaphoreType.DMA,
        pltpu.SemaphoreType.DMA,
        pltpu.SemaphoreType.DMA,
        pltpu.SemaphoreType.DMA,
        pltpu.SemaphoreType.DMA,
    ],
)
def _sc_embed(idx_hbm, table_hbm, pe_hbm, out_hbm,
              idx_v, r0, r1, r2, r3, p0, p1,
              gs0, gs1, gs2, gs3, os0, os1, os2, os3, ps0, ps1):
    wid = lax.axis_index("s") * _NC + lax.axis_index("c")
    pos0 = wid * _POS_PER_W

    rbuf = (r0, r1, r2, r3)
    pbuf = (p0, p1)
    gsem = (gs0, gs1, gs2, gs3)
    osem = (os0, os1, os2, os3)
    psem = (ps0, ps1)
    nbuf = 4

    pltpu.sync_copy(idx_hbm.at[wid], idx_v)

    def wait_gather(k):
        pltpu.make_async_copy(
            table_hbm.at[idx_v.at[0]], rbuf[k], gsem[k]).wait()

    def wait_out(k):
        pltpu.make_async_copy(
            rbuf[k], out_hbm.at[pl.ds(pos0, _CHUNK)], osem[k]).wait()

    def wait_pe(k):
        pltpu.make_async_copy(
            pe_hbm.at[pl.ds(pos0, _CHUNK)], pbuf[k], psem[k]).wait()

    pltpu.async_copy(pe_hbm.at[pl.ds(pos0, _CHUNK)], p0, ps0)
    for t in range(nbuf - 1):
        pltpu.async_copy(table_hbm.at[idx_v.at[t]], rbuf[t], gsem[t])

    def body(i2, _):
        for j in range(2 * _BATCH):
            s = 2 * _BATCH * i2 + j
            b = j % _BATCH
            cpar = (j // _BATCH) % 2
            c = 2 * i2 + j // _BATCH
            wait_gather(j % nbuf)
            if b == 0:
                wait_pe(cpar)

            rb = rbuf[j % nbuf]
            pb = pbuf[cpar]

            def _add_row(r, _):
                for k in range(_VECS_PER_ROW):
                    sl = pl.ds(k * 16, 16)
                    rb[r, sl] = rb[r, sl] + pb[r, sl]
                return 0

            lax.fori_loop(0, _CHUNK, _add_row, 0)
            pltpu.async_copy(
                rb,
                out_hbm.at[pl.ds(b * _SEQ_LEN + pos0 + c * _CHUNK, _CHUNK)],
                osem[j % nbuf])

            tj = j + nbuf - 1
            b1 = tj % _BATCH
            c1 = 2 * i2 + tj // _BATCH
            tpar = (tj // _BATCH) % 2

            def issue_ahead():
                if j == 0:
                    @pl.when(i2 >= 1)
                    def _():
                        wait_out(3)
                else:
                    wait_out((j - 1) % nbuf)
                if b1 == 0:
                    pltpu.async_copy(
                        pe_hbm.at[pl.ds(pos0 + c1 * _CHUNK, _CHUNK)],
                        pbuf[tpar], psem[tpar])
                pltpu.async_copy(
                    table_hbm.at[idx_v.at[s + nbuf - 1]],
                    rbuf[tj % nbuf], gsem[tj % nbuf])

            if j < 2 * _BATCH - (nbuf - 1):
                issue_ahead()
            else:
                @pl.when(i2 == 0)
                def _():
                    issue_ahead()
        return 0

    lax.fori_loop(0, _NSTEP // (2 * _BATCH), body, 0)

    for k in range(nbuf):
        wait_out(k)


def kernel(x, emb_table):
    batch, seq_len = x.shape
    d_model = emb_table.shape[1]
    idx = (x.astype(jnp.int32)
           .reshape(batch, _NW, _NCHUNK, _CHUNK)
           .transpose(1, 2, 0, 3)
           .reshape(_NW, _NSTEP, _CHUNK))
    out = _sc_embed(idx, emb_table, jnp.asarray(_PE))
    return out.reshape(batch, seq_len, d_model)
